# baseline (speedup 1.0000x reference)
import functools

import jax
import jax.numpy as jnp
from jax import lax
from jax.experimental import pallas as pl
from jax.experimental.pallas import tpu as pltpu
from jax.experimental.pallas import tpu_sc as plsc

N_TEAMS = 100000
N_CONFS = 1000
D = 128
B = 16384

NC = 2
NS = 16
NW = NC * NS
CHUNK = 128
N_BUF = 7
PREFETCH = 4
N_SPLIT = 2
NCP = 1024


def _sc_gather_body(n_chunks, id_row0,
                    team_off, team_def,
                    wt_ids, lt_ids,
                    t_wo, t_wd, t_lo, t_ld,
                    *scratch):
    rows = scratch[:N_BUF]
    idxs = scratch[N_BUF:N_BUF + 2]
    g_sems = scratch[N_BUF + 2:2 * N_BUF + 2]
    wb_sems = scratch[2 * N_BUF + 2:3 * N_BUF + 2]
    idx_sem = scratch[3 * N_BUF + 2]

    wid = lax.axis_index("s") * NC + lax.axis_index("c")
    base = wid * n_chunks * CHUNK

    idescs = [
        pltpu.async_copy(
            ids.at[pl.ds(id_row0 + wid * n_chunks, n_chunks)], idx, idx_sem)
        for ids, idx in zip((wt_ids, lt_ids), idxs)
    ]
    for dsc in idescs:
        dsc.wait()

    ops = []
    for c in range(n_chunks):
        off = base + c * CHUNK
        for idx, t_a, o_a, t_b, o_b in (
                (idxs[0], team_off, t_wo, team_def, t_wd),
                (idxs[1], team_off, t_lo, team_def, t_ld)):
            ops.append((idx.at[c], t_a, o_a, off))
            ops.append((idx.at[c], t_b, o_b, off))

    n_ops = len(ops)
    g_descs = [None] * n_ops
    wb_descs = [None] * n_ops

    def fire(j):
        b = j % N_BUF
        if j >= N_BUF and wb_descs[j - N_BUF] is not None:
            wb_descs[j - N_BUF].wait()
        idx, tbl = ops[j][0], ops[j][1]
        g_descs[j] = pltpu.async_copy(tbl.at[idx], rows[b], g_sems[b])

    for j in range(min(PREFETCH, n_ops)):
        fire(j)
    for j in range(n_ops):
        if j + PREFETCH < n_ops:
            fire(j + PREFETCH)
        b = j % N_BUF
        g_descs[j].wait()
        out, off = ops[j][2], ops[j][3]
        wb_descs[j] = pltpu.async_copy(
            rows[b], out.at[pl.ds(off, CHUNK)], wb_sems[b])
    for j in range(n_ops - N_BUF, n_ops):
        wb_descs[j].wait()


def _sc_gather(nb, h, team_off, team_def, wt_ids, lt_ids):
    n_chunks = nb // (NW * CHUNK)
    out = jax.ShapeDtypeStruct((nb, D), jnp.float32)
    mesh = plsc.VectorSubcoreMesh(core_axis_name="c", subcore_axis_name="s")
    return pl.kernel(
        functools.partial(_sc_gather_body, n_chunks, h * (nb // CHUNK)),
        out_type=[out] * 4,
        mesh=mesh,
        scratch_types=(
            [pltpu.VMEM((CHUNK, D), jnp.float32)] * N_BUF
            + [pltpu.VMEM((n_chunks, CHUNK), jnp.int32)] * 2
            + [pltpu.SemaphoreType.DMA] * (2 * N_BUF + 1)
        ),
    )(team_off, team_def, wt_ids, lt_ids)


BM = 2048


def _tc_mlp_body(t_wo, t_wd, t_lo, t_ld, wc_ids, lc_ids, conf_cat,
                 wloc, lloc, W1, b1, W2, b2, aw, ab, hw,
                 wscore, lscore):
    col = lax.broadcasted_iota(jnp.int32, (BM, NCP), 1)
    oh_w = (col == wc_ids[...]).astype(jnp.bfloat16)
    oh_l = (col == lc_ids[...]).astype(jnp.bfloat16)
    conf_w = jnp.dot(oh_w, conf_cat[...], preferred_element_type=jnp.float32)
    conf_l = jnp.dot(oh_l, conf_cat[...], preferred_element_type=jnp.float32)
    wo = t_wo[...] + conf_w[:, :D]
    wd = t_wd[...] + conf_w[:, D:]
    lo = t_lo[...] + conf_l[:, :D]
    ld = t_ld[...] + conf_l[:, D:]
    W1a = W1[:D, :]
    W1b = W1[D:, :]
    bias = b1[...]
    h_w = jnp.maximum(
        jnp.dot(wo, W1a, preferred_element_type=jnp.float32)
        + jnp.dot(ld, W1b, preferred_element_type=jnp.float32) + bias, 0.0)
    h_l = jnp.maximum(
        jnp.dot(lo, W1a, preferred_element_type=jnp.float32)
        + jnp.dot(wd, W1b, preferred_element_type=jnp.float32) + bias, 0.0)
    ws = jnp.dot(h_w, W2[...], preferred_element_type=jnp.float32) + b2[0, 0]
    ls = jnp.dot(h_l, W2[...], preferred_element_type=jnp.float32) + b2[0, 0]
    a_w = aw[0, 0]
    a_b = ab[0, 0]
    h_f = hw[0, 0]
    wscore[...] = ws * a_w + a_b + wloc[...] * h_f
    lscore[...] = ls * a_w + a_b + lloc[...] * h_f


def _tc_mlp(h, t_wo, t_wd, t_lo, t_ld, wc_ids, lc_ids, conf_cat,
            wloc, lloc, W1, b1, W2, b2, aw, ab, hw):
    nb = t_wo.shape[0]
    grid = (nb // BM,)
    off = h * (nb // BM)
    row_spec = pl.BlockSpec((BM, D), lambda i: (i, 0))
    col_spec = pl.BlockSpec((BM, 1), lambda i: (i, 0))
    col_off = pl.BlockSpec((BM, 1), lambda i: (i + off, 0))
    full = lambda shape: pl.BlockSpec(shape, lambda i: (0,) * len(shape))
    return pl.pallas_call(
        _tc_mlp_body,
        grid=grid,
        in_specs=[row_spec] * 4 + [col_off] * 2 + [full((NCP, 2 * D))]
        + [col_off] * 2 + [
            full((2 * D, D)), full((1, D)), full((D, 1)),
            full((1, 1)), full((1, 1)), full((1, 1)), full((1, 1)),
        ],
        out_specs=[col_spec, col_spec],
        out_shape=[jax.ShapeDtypeStruct((nb, 1), jnp.float32)] * 2,
    )(t_wo, t_wd, t_lo, t_ld, wc_ids, lc_ids, conf_cat,
      wloc, lloc, W1, b1, W2, b2, aw, ab, hw)


def kernel(team_offense, team_defense, conf_offense, conf_defense,
           winner_team_id, loser_team_id, winner_conf_id, loser_conf_id,
           winner_location, loser_location,
           W1, b1, W2, b2, affine_w, affine_b, home_w):
    nb = B // N_SPLIT
    conf_cat = jnp.zeros((NCP, 2 * D), dtype=jnp.bfloat16)
    conf_cat = lax.dynamic_update_slice(
        conf_cat,
        jnp.concatenate([conf_offense, conf_defense],
                        axis=1).astype(jnp.bfloat16),
        (0, 0))
    wt = winner_team_id.astype(jnp.int32).reshape(B // CHUNK, CHUNK)
    lt = loser_team_id.astype(jnp.int32).reshape(B // CHUNK, CHUNK)
    wc = winner_conf_id.astype(jnp.int32).reshape(B, 1)
    lc = loser_conf_id.astype(jnp.int32).reshape(B, 1)
    b1r = b1.reshape(1, D)
    b2r = b2.reshape(1, 1)
    abr = affine_b.reshape(1, 1)
    scores = []
    for h in range(N_SPLIT):
        t_wo, t_wd, t_lo, t_ld = _sc_gather(
            nb, h, team_offense, team_defense, wt, lt)
        scores.append(_tc_mlp(
            h, t_wo, t_wd, t_lo, t_ld, wc, lc, conf_cat,
            winner_location, loser_location,
            W1, b1r, W2, b2r, affine_w, abr, home_w))
    wscore = jnp.concatenate([s[0] for s in scores], axis=0)
    lscore = jnp.concatenate([s[1] for s in scores], axis=0)
    return (wscore, lscore)


# speedup vs baseline: 1.0085x; 1.0085x over previous
import functools

import jax
import jax.numpy as jnp
from jax import lax
from jax.experimental import pallas as pl
from jax.experimental.pallas import tpu as pltpu
from jax.experimental.pallas import tpu_sc as plsc

N_TEAMS = 100000
N_CONFS = 1000
D = 128
B = 16384

NC = 2
NS = 16
NW = NC * NS
CHUNK = 128
N_BUF = 7
PREFETCH = 4
N_SPLIT = 1
NCP = 1024


def _sc_gather_body(n_chunks, id_row0,
                    team_off, team_def,
                    wt_ids, lt_ids,
                    t_wo, t_wd, t_lo, t_ld,
                    *scratch):
    rows = scratch[:N_BUF]
    idxs = scratch[N_BUF:N_BUF + 2]
    g_sems = scratch[N_BUF + 2:2 * N_BUF + 2]
    wb_sems = scratch[2 * N_BUF + 2:3 * N_BUF + 2]
    idx_sem = scratch[3 * N_BUF + 2]

    wid = lax.axis_index("s") * NC + lax.axis_index("c")
    base = wid * n_chunks * CHUNK

    idescs = [
        pltpu.async_copy(
            ids.at[pl.ds(id_row0 + wid * n_chunks, n_chunks)], idx, idx_sem)
        for ids, idx in zip((wt_ids, lt_ids), idxs)
    ]
    for dsc in idescs:
        dsc.wait()

    ops = []
    for c in range(n_chunks):
        off = base + c * CHUNK
        for idx, t_a, o_a, t_b, o_b in (
                (idxs[0], team_off, t_wo, team_def, t_wd),
                (idxs[1], team_off, t_lo, team_def, t_ld)):
            ops.append((idx.at[c], t_a, o_a, off))
            ops.append((idx.at[c], t_b, o_b, off))

    n_ops = len(ops)
    g_descs = [None] * n_ops
    wb_descs = [None] * n_ops

    def fire(j):
        b = j % N_BUF
        if j >= N_BUF and wb_descs[j - N_BUF] is not None:
            wb_descs[j - N_BUF].wait()
        idx, tbl = ops[j][0], ops[j][1]
        g_descs[j] = pltpu.async_copy(tbl.at[idx], rows[b], g_sems[b])

    for j in range(min(PREFETCH, n_ops)):
        fire(j)
    for j in range(n_ops):
        if j + PREFETCH < n_ops:
            fire(j + PREFETCH)
        b = j % N_BUF
        g_descs[j].wait()
        out, off = ops[j][2], ops[j][3]
        wb_descs[j] = pltpu.async_copy(
            rows[b], out.at[pl.ds(off, CHUNK)], wb_sems[b])
    for j in range(n_ops - N_BUF, n_ops):
        wb_descs[j].wait()


def _sc_gather(nb, h, team_off, team_def, wt_ids, lt_ids):
    n_chunks = nb // (NW * CHUNK)
    out = jax.ShapeDtypeStruct((nb, D), jnp.float32)
    mesh = plsc.VectorSubcoreMesh(core_axis_name="c", subcore_axis_name="s")
    return pl.kernel(
        functools.partial(_sc_gather_body, n_chunks, h * (nb // CHUNK)),
        out_type=[out] * 4,
        mesh=mesh,
        scratch_types=(
            [pltpu.VMEM((CHUNK, D), jnp.float32)] * N_BUF
            + [pltpu.VMEM((n_chunks, CHUNK), jnp.int32)] * 2
            + [pltpu.SemaphoreType.DMA] * (2 * N_BUF + 1)
        ),
    )(team_off, team_def, wt_ids, lt_ids)


BM = 1024


def _tc_mlp_body(t_wo, t_wd, t_lo, t_ld, wc_ids, lc_ids, conf_cat,
                 wloc, lloc, W1, b1, W2, b2, aw, ab, hw,
                 wscore, lscore):
    col = lax.broadcasted_iota(jnp.int32, (BM, NCP), 1)
    oh_w = (col == wc_ids[...]).astype(jnp.bfloat16)
    oh_l = (col == lc_ids[...]).astype(jnp.bfloat16)
    conf_w = jnp.dot(oh_w, conf_cat[...], preferred_element_type=jnp.float32)
    conf_l = jnp.dot(oh_l, conf_cat[...], preferred_element_type=jnp.float32)
    wo = t_wo[...] + conf_w[:, :D]
    wd = t_wd[...] + conf_w[:, D:]
    lo = t_lo[...] + conf_l[:, :D]
    ld = t_ld[...] + conf_l[:, D:]
    W1a = W1[:D, :]
    W1b = W1[D:, :]
    bias = b1[...]
    h_w = jnp.maximum(
        jnp.dot(wo, W1a, preferred_element_type=jnp.float32)
        + jnp.dot(ld, W1b, preferred_element_type=jnp.float32) + bias, 0.0)
    h_l = jnp.maximum(
        jnp.dot(lo, W1a, preferred_element_type=jnp.float32)
        + jnp.dot(wd, W1b, preferred_element_type=jnp.float32) + bias, 0.0)
    ws = jnp.dot(h_w, W2[...], preferred_element_type=jnp.float32) + b2[0, 0]
    ls = jnp.dot(h_l, W2[...], preferred_element_type=jnp.float32) + b2[0, 0]
    a_w = aw[0, 0]
    a_b = ab[0, 0]
    h_f = hw[0, 0]
    wscore[...] = ws * a_w + a_b + wloc[...] * h_f
    lscore[...] = ls * a_w + a_b + lloc[...] * h_f


def _tc_mlp(h, t_wo, t_wd, t_lo, t_ld, wc_ids, lc_ids, conf_cat,
            wloc, lloc, W1, b1, W2, b2, aw, ab, hw):
    nb = t_wo.shape[0]
    grid = (nb // BM,)
    off = h * (nb // BM)
    row_spec = pl.BlockSpec((BM, D), lambda i: (i, 0))
    col_spec = pl.BlockSpec((BM, 1), lambda i: (i, 0))
    col_off = pl.BlockSpec((BM, 1), lambda i: (i + off, 0))
    full = lambda shape: pl.BlockSpec(shape, lambda i: (0,) * len(shape))
    return pl.pallas_call(
        _tc_mlp_body,
        grid=grid,
        in_specs=[row_spec] * 4 + [col_off] * 2 + [full((NCP, 2 * D))]
        + [col_off] * 2 + [
            full((2 * D, D)), full((1, D)), full((D, 1)),
            full((1, 1)), full((1, 1)), full((1, 1)), full((1, 1)),
        ],
        out_specs=[col_spec, col_spec],
        out_shape=[jax.ShapeDtypeStruct((nb, 1), jnp.float32)] * 2,
    )(t_wo, t_wd, t_lo, t_ld, wc_ids, lc_ids, conf_cat,
      wloc, lloc, W1, b1, W2, b2, aw, ab, hw)


def kernel(team_offense, team_defense, conf_offense, conf_defense,
           winner_team_id, loser_team_id, winner_conf_id, loser_conf_id,
           winner_location, loser_location,
           W1, b1, W2, b2, affine_w, affine_b, home_w):
    nb = B // N_SPLIT
    conf_cat = jnp.zeros((NCP, 2 * D), dtype=jnp.bfloat16)
    conf_cat = lax.dynamic_update_slice(
        conf_cat,
        jnp.concatenate([conf_offense, conf_defense],
                        axis=1).astype(jnp.bfloat16),
        (0, 0))
    wt = winner_team_id.astype(jnp.int32).reshape(B // CHUNK, CHUNK)
    lt = loser_team_id.astype(jnp.int32).reshape(B // CHUNK, CHUNK)
    wc = winner_conf_id.astype(jnp.int32).reshape(B, 1)
    lc = loser_conf_id.astype(jnp.int32).reshape(B, 1)
    b1r = b1.reshape(1, D)
    b2r = b2.reshape(1, 1)
    abr = affine_b.reshape(1, 1)
    scores = []
    for h in range(N_SPLIT):
        t_wo, t_wd, t_lo, t_ld = _sc_gather(
            nb, h, team_offense, team_defense, wt, lt)
        scores.append(_tc_mlp(
            h, t_wo, t_wd, t_lo, t_ld, wc, lc, conf_cat,
            winner_location, loser_location,
            W1, b1r, W2, b2r, affine_w, abr, home_w))
    wscore = jnp.concatenate([s[0] for s in scores], axis=0)
    lscore = jnp.concatenate([s[1] for s in scores], axis=0)
    return (wscore, lscore)


# speedup vs baseline: 1.1737x; 1.1638x over previous
import functools

import jax
import jax.numpy as jnp
from jax import lax
from jax.experimental import pallas as pl
from jax.experimental.pallas import tpu as pltpu
from jax.experimental.pallas import tpu_sc as plsc

N_TEAMS = 100000
N_CONFS = 1000
D = 128
B = 16384

NC = 2
NS = 16
NW = NC * NS
CHUNK = 128
N_BUF = 7
PREFETCH = 4
N_SPLIT = 1
NCP = 1024


def _sc_gather_body(n_chunks, id_row0,
                    team_off, team_def,
                    wt_ids, lt_ids,
                    t_wo, t_wd, t_lo, t_ld,
                    *scratch):
    rows = scratch[:N_BUF]
    idxs = scratch[N_BUF:N_BUF + 2]
    g_sems = scratch[N_BUF + 2:2 * N_BUF + 2]
    wb_sems = scratch[2 * N_BUF + 2:3 * N_BUF + 2]
    idx_sem = scratch[3 * N_BUF + 2]

    wid = lax.axis_index("s") * NC + lax.axis_index("c")
    base = wid * n_chunks * CHUNK

    idescs = [
        pltpu.async_copy(
            ids.at[pl.ds(id_row0 + wid * n_chunks, n_chunks)], idx, idx_sem)
        for ids, idx in zip((wt_ids, lt_ids), idxs)
    ]
    for dsc in idescs:
        dsc.wait()

    ops = []
    for c in range(n_chunks):
        off = base + c * CHUNK
        for idx, t_a, o_a, t_b, o_b in (
                (idxs[0], team_off, t_wo, team_def, t_wd),
                (idxs[1], team_off, t_lo, team_def, t_ld)):
            ops.append((idx.at[c], t_a, o_a, off))
            ops.append((idx.at[c], t_b, o_b, off))

    n_ops = len(ops)
    g_descs = [None] * n_ops
    wb_descs = [None] * n_ops

    def fire(j):
        b = j % N_BUF
        if j >= N_BUF and wb_descs[j - N_BUF] is not None:
            wb_descs[j - N_BUF].wait()
        idx, tbl = ops[j][0], ops[j][1]
        g_descs[j] = pltpu.async_copy(tbl.at[idx], rows[b], g_sems[b])

    for j in range(min(PREFETCH, n_ops)):
        fire(j)
    for j in range(n_ops):
        if j + PREFETCH < n_ops:
            fire(j + PREFETCH)
        b = j % N_BUF
        g_descs[j].wait()
        out, off = ops[j][2], ops[j][3]
        wb_descs[j] = pltpu.async_copy(
            rows[b], out.at[pl.ds(off, CHUNK)], wb_sems[b])
    for j in range(n_ops - N_BUF, n_ops):
        wb_descs[j].wait()


def _sc_gather(nb, h, team_off, team_def, wt_ids, lt_ids):
    n_chunks = nb // (NW * CHUNK)
    out = jax.ShapeDtypeStruct((nb, D), jnp.float32)
    mesh = plsc.VectorSubcoreMesh(core_axis_name="c", subcore_axis_name="s")
    return pl.kernel(
        functools.partial(_sc_gather_body, n_chunks, h * (nb // CHUNK)),
        out_type=[out] * 4,
        mesh=mesh,
        scratch_types=(
            [pltpu.VMEM((CHUNK, D), jnp.float32)] * N_BUF
            + [pltpu.VMEM((n_chunks, CHUNK), jnp.int32)] * 2
            + [pltpu.SemaphoreType.DMA] * (2 * N_BUF + 1)
        ),
    )(team_off, team_def, wt_ids, lt_ids)


BM = 1024


def _tc_mlp_body(t_wo, t_wd, t_lo, t_ld, wc_ids, lc_ids, conf_cat,
                 wloc, lloc, W1, b1, W2r, b2, aw, ab, hw,
                 wscore, lscore):
    col = lax.broadcasted_iota(jnp.int32, (BM, NCP), 1)
    oh_w = (col == wc_ids[...].reshape(BM, 1)).astype(jnp.bfloat16)
    oh_l = (col == lc_ids[...].reshape(BM, 1)).astype(jnp.bfloat16)
    conf_w = jnp.dot(oh_w, conf_cat[...], preferred_element_type=jnp.float32)
    conf_l = jnp.dot(oh_l, conf_cat[...], preferred_element_type=jnp.float32)
    wo = t_wo[...] + conf_w[:, :D]
    wd = t_wd[...] + conf_w[:, D:]
    lo = t_lo[...] + conf_l[:, :D]
    ld = t_ld[...] + conf_l[:, D:]
    W1a = W1[:D, :]
    W1b = W1[D:, :]
    bias = b1[...]
    h_w = jnp.maximum(
        jnp.dot(wo, W1a, preferred_element_type=jnp.float32)
        + jnp.dot(ld, W1b, preferred_element_type=jnp.float32) + bias, 0.0)
    h_l = jnp.maximum(
        jnp.dot(lo, W1a, preferred_element_type=jnp.float32)
        + jnp.dot(wd, W1b, preferred_element_type=jnp.float32) + bias, 0.0)
    w2row = W2r[...]
    ws = jnp.sum(h_w * w2row, axis=1).reshape(1, BM) + b2[0, 0]
    ls = jnp.sum(h_l * w2row, axis=1).reshape(1, BM) + b2[0, 0]
    a_w = aw[0, 0]
    a_b = ab[0, 0]
    h_f = hw[0, 0]
    wscore[...] = ws * a_w + a_b + wloc[...] * h_f
    lscore[...] = ls * a_w + a_b + lloc[...] * h_f


def _tc_mlp(h, t_wo, t_wd, t_lo, t_ld, wc_ids, lc_ids, conf_cat,
            wloc, lloc, W1, b1, W2, b2, aw, ab, hw):
    nb = t_wo.shape[0]
    grid = (nb // BM,)
    off = h * (nb // BM)
    row_spec = pl.BlockSpec((BM, D), lambda i: (i, 0))
    vec_spec = pl.BlockSpec((1, BM), lambda i: (0, i))
    vec_off = pl.BlockSpec((1, BM), lambda i: (0, i + off))
    full = lambda shape: pl.BlockSpec(shape, lambda i: (0,) * len(shape))
    return pl.pallas_call(
        _tc_mlp_body,
        grid=grid,
        in_specs=[row_spec] * 4 + [vec_off] * 2 + [full((NCP, 2 * D))]
        + [vec_off] * 2 + [
            full((2 * D, D)), full((1, D)), full((1, D)),
            full((1, 1)), full((1, 1)), full((1, 1)), full((1, 1)),
        ],
        out_specs=[vec_spec, vec_spec],
        out_shape=[jax.ShapeDtypeStruct((1, nb), jnp.float32)] * 2,
    )(t_wo, t_wd, t_lo, t_ld, wc_ids, lc_ids, conf_cat,
      wloc, lloc, W1, b1, W2, b2, aw, ab, hw)


def kernel(team_offense, team_defense, conf_offense, conf_defense,
           winner_team_id, loser_team_id, winner_conf_id, loser_conf_id,
           winner_location, loser_location,
           W1, b1, W2, b2, affine_w, affine_b, home_w):
    nb = B // N_SPLIT
    conf_cat = jnp.zeros((NCP, 2 * D), dtype=jnp.bfloat16)
    conf_cat = lax.dynamic_update_slice(
        conf_cat,
        jnp.concatenate([conf_offense, conf_defense],
                        axis=1).astype(jnp.bfloat16),
        (0, 0))
    wt = winner_team_id.astype(jnp.int32).reshape(B // CHUNK, CHUNK)
    lt = loser_team_id.astype(jnp.int32).reshape(B // CHUNK, CHUNK)
    wc = winner_conf_id.astype(jnp.int32).reshape(1, B)
    lc = loser_conf_id.astype(jnp.int32).reshape(1, B)
    wlocr = winner_location.reshape(1, B)
    llocr = loser_location.reshape(1, B)
    b1r = b1.reshape(1, D)
    b2r = b2.reshape(1, 1)
    abr = affine_b.reshape(1, 1)
    w2r = W2.reshape(1, D)
    scores = []
    for h in range(N_SPLIT):
        t_wo, t_wd, t_lo, t_ld = _sc_gather(
            nb, h, team_offense, team_defense, wt, lt)
        scores.append(_tc_mlp(
            h, t_wo, t_wd, t_lo, t_ld, wc, lc, conf_cat,
            wlocr, llocr,
            W1, b1r, w2r, b2r, affine_w, abr, home_w))
    wscore = jnp.concatenate([s[0] for s in scores], axis=1).reshape(B, 1)
    lscore = jnp.concatenate([s[1] for s in scores], axis=1).reshape(B, 1)
    return (wscore, lscore)


# speedup vs baseline: 1.2479x; 1.0632x over previous
import functools

import jax
import jax.numpy as jnp
from jax import lax
from jax.experimental import pallas as pl
from jax.experimental.pallas import tpu as pltpu
from jax.experimental.pallas import tpu_sc as plsc

N_TEAMS = 100000
N_CONFS = 1000
D = 128
B = 16384

NC = 2
NS = 16
NW = NC * NS
CHUNK = 128
N_BUF = 7
PREFETCH = 4
N_SPLIT = 2
NCP = 1024


def _sc_gather_body(n_chunks, id_row0,
                    team_off, team_def,
                    wt_ids, lt_ids,
                    t_wo, t_wd, t_lo, t_ld,
                    *scratch):
    rows = scratch[:N_BUF]
    idxs = scratch[N_BUF:N_BUF + 2]
    g_sems = scratch[N_BUF + 2:2 * N_BUF + 2]
    wb_sems = scratch[2 * N_BUF + 2:3 * N_BUF + 2]
    idx_sem = scratch[3 * N_BUF + 2]

    wid = lax.axis_index("s") * NC + lax.axis_index("c")
    base = wid * n_chunks * CHUNK

    idescs = [
        pltpu.async_copy(
            ids.at[pl.ds(id_row0 + wid * n_chunks, n_chunks)], idx, idx_sem)
        for ids, idx in zip((wt_ids, lt_ids), idxs)
    ]
    for dsc in idescs:
        dsc.wait()

    ops = []
    for c in range(n_chunks):
        off = base + c * CHUNK
        for idx, t_a, o_a, t_b, o_b in (
                (idxs[0], team_off, t_wo, team_def, t_wd),
                (idxs[1], team_off, t_lo, team_def, t_ld)):
            ops.append((idx.at[c], t_a, o_a, off))
            ops.append((idx.at[c], t_b, o_b, off))

    n_ops = len(ops)
    g_descs = [None] * n_ops
    wb_descs = [None] * n_ops

    def fire(j):
        b = j % N_BUF
        if j >= N_BUF and wb_descs[j - N_BUF] is not None:
            wb_descs[j - N_BUF].wait()
        idx, tbl = ops[j][0], ops[j][1]
        g_descs[j] = pltpu.async_copy(tbl.at[idx], rows[b], g_sems[b])

    for j in range(min(PREFETCH, n_ops)):
        fire(j)
    for j in range(n_ops):
        if j + PREFETCH < n_ops:
            fire(j + PREFETCH)
        b = j % N_BUF
        g_descs[j].wait()
        out, off = ops[j][2], ops[j][3]
        wb_descs[j] = pltpu.async_copy(
            rows[b], out.at[pl.ds(off, CHUNK)], wb_sems[b])
    for j in range(n_ops - N_BUF, n_ops):
        wb_descs[j].wait()


def _sc_gather(nb, h, team_off, team_def, wt_ids, lt_ids):
    n_chunks = nb // (NW * CHUNK)
    out = jax.ShapeDtypeStruct((nb, D), jnp.float32)
    mesh = plsc.VectorSubcoreMesh(core_axis_name="c", subcore_axis_name="s")
    return pl.kernel(
        functools.partial(_sc_gather_body, n_chunks, h * (nb // CHUNK)),
        out_type=[out] * 4,
        mesh=mesh,
        scratch_types=(
            [pltpu.VMEM((CHUNK, D), jnp.float32)] * N_BUF
            + [pltpu.VMEM((n_chunks, CHUNK), jnp.int32)] * 2
            + [pltpu.SemaphoreType.DMA] * (2 * N_BUF + 1)
        ),
    )(team_off, team_def, wt_ids, lt_ids)


BM = 1024


def _tc_mlp_body(t_wo, t_wd, t_lo, t_ld, wc_ids, lc_ids, conf_cat,
                 wloc, lloc, W1, b1, W2r, b2, aw, ab, hw,
                 wscore, lscore):
    col = lax.broadcasted_iota(jnp.int32, (BM, NCP), 1)
    oh_w = (col == wc_ids[...].reshape(BM, 1)).astype(jnp.bfloat16)
    oh_l = (col == lc_ids[...].reshape(BM, 1)).astype(jnp.bfloat16)
    conf_w = jnp.dot(oh_w, conf_cat[...], preferred_element_type=jnp.float32)
    conf_l = jnp.dot(oh_l, conf_cat[...], preferred_element_type=jnp.float32)
    wo = t_wo[...] + conf_w[:, :D]
    wd = t_wd[...] + conf_w[:, D:]
    lo = t_lo[...] + conf_l[:, :D]
    ld = t_ld[...] + conf_l[:, D:]
    W1a = W1[:D, :]
    W1b = W1[D:, :]
    bias = b1[...]
    h_w = jnp.maximum(
        jnp.dot(wo, W1a, preferred_element_type=jnp.float32)
        + jnp.dot(ld, W1b, preferred_element_type=jnp.float32) + bias, 0.0)
    h_l = jnp.maximum(
        jnp.dot(lo, W1a, preferred_element_type=jnp.float32)
        + jnp.dot(wd, W1b, preferred_element_type=jnp.float32) + bias, 0.0)
    w2row = W2r[...]
    ws = jnp.sum(h_w * w2row, axis=1).reshape(1, BM) + b2[0, 0]
    ls = jnp.sum(h_l * w2row, axis=1).reshape(1, BM) + b2[0, 0]
    a_w = aw[0, 0]
    a_b = ab[0, 0]
    h_f = hw[0, 0]
    wscore[...] = ws * a_w + a_b + wloc[...] * h_f
    lscore[...] = ls * a_w + a_b + lloc[...] * h_f


def _tc_mlp(h, t_wo, t_wd, t_lo, t_ld, wc_ids, lc_ids, conf_cat,
            wloc, lloc, W1, b1, W2, b2, aw, ab, hw):
    nb = t_wo.shape[0]
    grid = (nb // BM,)
    off = h * (nb // BM)
    row_spec = pl.BlockSpec((BM, D), lambda i: (i, 0))
    vec_spec = pl.BlockSpec((1, BM), lambda i: (0, i))
    vec_off = pl.BlockSpec((1, BM), lambda i: (0, i + off))
    full = lambda shape: pl.BlockSpec(shape, lambda i: (0,) * len(shape))
    return pl.pallas_call(
        _tc_mlp_body,
        grid=grid,
        in_specs=[row_spec] * 4 + [vec_off] * 2 + [full((NCP, 2 * D))]
        + [vec_off] * 2 + [
            full((2 * D, D)), full((1, D)), full((1, D)),
            full((1, 1)), full((1, 1)), full((1, 1)), full((1, 1)),
        ],
        out_specs=[vec_spec, vec_spec],
        out_shape=[jax.ShapeDtypeStruct((1, nb), jnp.float32)] * 2,
    )(t_wo, t_wd, t_lo, t_ld, wc_ids, lc_ids, conf_cat,
      wloc, lloc, W1, b1, W2, b2, aw, ab, hw)


def kernel(team_offense, team_defense, conf_offense, conf_defense,
           winner_team_id, loser_team_id, winner_conf_id, loser_conf_id,
           winner_location, loser_location,
           W1, b1, W2, b2, affine_w, affine_b, home_w):
    nb = B // N_SPLIT
    conf_cat = jnp.zeros((NCP, 2 * D), dtype=jnp.bfloat16)
    conf_cat = lax.dynamic_update_slice(
        conf_cat,
        jnp.concatenate([conf_offense, conf_defense],
                        axis=1).astype(jnp.bfloat16),
        (0, 0))
    wt = winner_team_id.astype(jnp.int32).reshape(B // CHUNK, CHUNK)
    lt = loser_team_id.astype(jnp.int32).reshape(B // CHUNK, CHUNK)
    wc = winner_conf_id.astype(jnp.int32).reshape(1, B)
    lc = loser_conf_id.astype(jnp.int32).reshape(1, B)
    wlocr = winner_location.reshape(1, B)
    llocr = loser_location.reshape(1, B)
    b1r = b1.reshape(1, D)
    b2r = b2.reshape(1, 1)
    abr = affine_b.reshape(1, 1)
    w2r = W2.reshape(1, D)
    scores = []
    for h in range(N_SPLIT):
        t_wo, t_wd, t_lo, t_ld = _sc_gather(
            nb, h, team_offense, team_defense, wt, lt)
        scores.append(_tc_mlp(
            h, t_wo, t_wd, t_lo, t_ld, wc, lc, conf_cat,
            wlocr, llocr,
            W1, b1r, w2r, b2r, affine_w, abr, home_w))
    wscore = jnp.concatenate([s[0] for s in scores], axis=1).reshape(B, 1)
    lscore = jnp.concatenate([s[1] for s in scores], axis=1).reshape(B, 1)
    return (wscore, lscore)


# speedup vs baseline: 1.4682x; 1.1766x over previous
import functools

import jax
import jax.numpy as jnp
from jax import lax
from jax.experimental import pallas as pl
from jax.experimental.pallas import tpu as pltpu
from jax.experimental.pallas import tpu_sc as plsc

N_TEAMS = 100000
N_CONFS = 1000
D = 128
B = 16384

NC = 2
NS = 16
NW = NC * NS
CHUNK = 128
N_BUF = 7
PREFETCH = 4
N_SPLIT = 2
NCP = 1024


def _sc_gather_body(n_chunks, id_row0,
                    team_off, team_def,
                    wt_ids, lt_ids,
                    t_wo, t_wd, t_lo, t_ld,
                    *scratch):
    rows = scratch[:N_BUF]
    idxs = scratch[N_BUF:N_BUF + 2]
    g_sems = scratch[N_BUF + 2:2 * N_BUF + 2]
    wb_sems = scratch[2 * N_BUF + 2:3 * N_BUF + 2]
    idx_sem = scratch[3 * N_BUF + 2]

    wid = lax.axis_index("s") * NC + lax.axis_index("c")
    base = wid * n_chunks * CHUNK

    idescs = [
        pltpu.async_copy(
            ids.at[pl.ds(id_row0 + wid * n_chunks, n_chunks)], idx, idx_sem)
        for ids, idx in zip((wt_ids, lt_ids), idxs)
    ]
    for dsc in idescs:
        dsc.wait()

    ops = []
    for c in range(n_chunks):
        off = base + c * CHUNK
        for idx, t_a, o_a, t_b, o_b in (
                (idxs[0], team_off, t_wo, team_def, t_wd),
                (idxs[1], team_off, t_lo, team_def, t_ld)):
            ops.append((idx.at[c], t_a, o_a, off))
            ops.append((idx.at[c], t_b, o_b, off))

    n_ops = len(ops)
    g_descs = [None] * n_ops
    wb_descs = [None] * n_ops

    def fire(j):
        b = j % N_BUF
        if j >= N_BUF and wb_descs[j - N_BUF] is not None:
            wb_descs[j - N_BUF].wait()
        idx, tbl = ops[j][0], ops[j][1]
        g_descs[j] = pltpu.async_copy(tbl.at[idx], rows[b], g_sems[b])

    for j in range(min(PREFETCH, n_ops)):
        fire(j)
    for j in range(n_ops):
        if j + PREFETCH < n_ops:
            fire(j + PREFETCH)
        b = j % N_BUF
        g_descs[j].wait()
        out, off = ops[j][2], ops[j][3]
        wb_descs[j] = pltpu.async_copy(
            rows[b], out.at[pl.ds(off, CHUNK)], wb_sems[b])
    for j in range(n_ops - N_BUF, n_ops):
        wb_descs[j].wait()


def _sc_gather(nb, h, team_off, team_def, wt_ids, lt_ids):
    n_chunks = nb // (NW * CHUNK)
    out = jax.ShapeDtypeStruct((nb, D), jnp.float32)
    mesh = plsc.VectorSubcoreMesh(core_axis_name="c", subcore_axis_name="s")
    return pl.kernel(
        functools.partial(_sc_gather_body, n_chunks, h * (nb // CHUNK)),
        out_type=[out] * 4,
        mesh=mesh,
        scratch_types=(
            [pltpu.VMEM((CHUNK, D), jnp.float32)] * N_BUF
            + [pltpu.VMEM((n_chunks, CHUNK), jnp.int32)] * 2
            + [pltpu.SemaphoreType.DMA] * (2 * N_BUF + 1)
        ),
    )(team_off, team_def, wt_ids, lt_ids)


BM = 1024


def _tc_mlp_body(t_wo, t_wd, t_lo, t_ld, wc_ids, lc_ids, conf_cat,
                 wloc, lloc, W1, b1, W2r, b2, aw, ab, hw,
                 wscore, lscore):
    col = lax.broadcasted_iota(jnp.int32, (BM, NCP), 1)
    oh_w = (col == wc_ids[...].reshape(BM, 1)).astype(jnp.bfloat16)
    oh_l = (col == lc_ids[...].reshape(BM, 1)).astype(jnp.bfloat16)
    conf_w = jnp.dot(oh_w, conf_cat[...], preferred_element_type=jnp.float32)
    conf_l = jnp.dot(oh_l, conf_cat[...], preferred_element_type=jnp.float32)
    wo = t_wo[...] + conf_w[:, :D]
    wd = t_wd[...] + conf_w[:, D:]
    lo = t_lo[...] + conf_l[:, :D]
    ld = t_ld[...] + conf_l[:, D:]
    W1a = W1[:D, :]
    W1b = W1[D:, :]
    bias = b1[...]
    h_w = jnp.maximum(
        jnp.dot(wo, W1a, preferred_element_type=jnp.float32)
        + jnp.dot(ld, W1b, preferred_element_type=jnp.float32) + bias, 0.0)
    h_l = jnp.maximum(
        jnp.dot(lo, W1a, preferred_element_type=jnp.float32)
        + jnp.dot(wd, W1b, preferred_element_type=jnp.float32) + bias, 0.0)
    dn = (((1,), (1,)), ((), ()))
    w2row = W2r[...]
    ws = lax.dot_general(w2row, h_w, dn,
                         preferred_element_type=jnp.float32) + b2[0, 0]
    ls = lax.dot_general(w2row, h_l, dn,
                         preferred_element_type=jnp.float32) + b2[0, 0]
    a_w = aw[0, 0]
    a_b = ab[0, 0]
    h_f = hw[0, 0]
    wscore[...] = ws * a_w + a_b + wloc[...] * h_f
    lscore[...] = ls * a_w + a_b + lloc[...] * h_f


def _tc_mlp(h, t_wo, t_wd, t_lo, t_ld, wc_ids, lc_ids, conf_cat,
            wloc, lloc, W1, b1, W2, b2, aw, ab, hw):
    nb = t_wo.shape[0]
    grid = (nb // BM,)
    off = h * (nb // BM)
    row_spec = pl.BlockSpec((BM, D), lambda i: (i, 0))
    vec_spec = pl.BlockSpec((1, BM), lambda i: (0, i))
    vec_off = pl.BlockSpec((1, BM), lambda i: (0, i + off))
    full = lambda shape: pl.BlockSpec(shape, lambda i: (0,) * len(shape))
    return pl.pallas_call(
        _tc_mlp_body,
        grid=grid,
        in_specs=[row_spec] * 4 + [vec_off] * 2 + [full((NCP, 2 * D))]
        + [vec_off] * 2 + [
            full((2 * D, D)), full((1, D)), full((1, D)),
            full((1, 1)), full((1, 1)), full((1, 1)), full((1, 1)),
        ],
        out_specs=[vec_spec, vec_spec],
        out_shape=[jax.ShapeDtypeStruct((1, nb), jnp.float32)] * 2,
    )(t_wo, t_wd, t_lo, t_ld, wc_ids, lc_ids, conf_cat,
      wloc, lloc, W1, b1, W2, b2, aw, ab, hw)


def kernel(team_offense, team_defense, conf_offense, conf_defense,
           winner_team_id, loser_team_id, winner_conf_id, loser_conf_id,
           winner_location, loser_location,
           W1, b1, W2, b2, affine_w, affine_b, home_w):
    nb = B // N_SPLIT
    conf_cat = jnp.zeros((NCP, 2 * D), dtype=jnp.bfloat16)
    conf_cat = lax.dynamic_update_slice(
        conf_cat,
        jnp.concatenate([conf_offense, conf_defense],
                        axis=1).astype(jnp.bfloat16),
        (0, 0))
    wt = winner_team_id.astype(jnp.int32).reshape(B // CHUNK, CHUNK)
    lt = loser_team_id.astype(jnp.int32).reshape(B // CHUNK, CHUNK)
    wc = winner_conf_id.astype(jnp.int32).reshape(1, B)
    lc = loser_conf_id.astype(jnp.int32).reshape(1, B)
    wlocr = winner_location.reshape(1, B)
    llocr = loser_location.reshape(1, B)
    b1r = b1.reshape(1, D)
    b2r = b2.reshape(1, 1)
    abr = affine_b.reshape(1, 1)
    w2r = W2.reshape(1, D)
    scores = []
    for h in range(N_SPLIT):
        t_wo, t_wd, t_lo, t_ld = _sc_gather(
            nb, h, team_offense, team_defense, wt, lt)
        scores.append(_tc_mlp(
            h, t_wo, t_wd, t_lo, t_ld, wc, lc, conf_cat,
            wlocr, llocr,
            W1, b1r, w2r, b2r, affine_w, abr, home_w))
    wscore = jnp.concatenate([s[0] for s in scores], axis=1).reshape(B, 1)
    lscore = jnp.concatenate([s[1] for s in scores], axis=1).reshape(B, 1)
    return (wscore, lscore)
